# Initial kernel scaffold; baseline (speedup 1.0000x reference)
import jax
import jax.numpy as jnp
from jax.experimental import pallas as pl


def kernel(x, targets):
    raise NotImplementedError("write your pallas kernel here")


# speedup vs baseline: 49.8577x; 49.8577x over previous
import functools

import jax
import jax.numpy as jnp
import numpy as np
from jax import lax
from jax.experimental import pallas as pl
from jax.experimental.pallas import tpu as pltpu
from jax.experimental.pallas import tpu_sc as plsc

_ANCHORS = np.array(
    [[1.3221, 1.73145], [3.19275, 4.00944], [5.05587, 8.09892],
     [9.47112, 4.84053], [11.2364, 10.0071]], dtype=np.float32)
_THRESH = 0.6
_GH, _GW = 16, 32
_NA, _NCLS = 5, 7
_B, _T = 64, 50
_CELLS = _NA * _GH * _GW
_HW = _GH * _GW
_SC_CORES, _SC_SUBCORES = 2, 16
_NW = _SC_CORES * _SC_SUBCORES
_TPAD = 256


def _sc_body(tgt_hbm, mask_o, conf_o, tx_o, ty_o, rw_o, rh_o, lab_o,
             tgt_v, mask_t, conf_t, tx_t, ty_t, rw_t, rh_t, lab_t):
  wid = lax.axis_index("s") * _SC_CORES + lax.axis_index("c")
  lanes = lax.iota(jnp.int32, 16)
  in5 = lanes < _NA
  zf = jnp.zeros((16,), jnp.float32)

  def _lanes_const(vals):
    v = zf
    for i, c in enumerate(vals):
      v = jnp.where(lanes == i, float(c), v)
    return v

  aw = _lanes_const(_ANCHORS[:, 0])
  ah = _lanes_const(_ANCHORS[:, 1])

  for b_off in (0, _NW):
    b = wid + b_off
    pltpu.sync_copy(tgt_hbm.at[b], tgt_v)

    def init_body(i, c):
      sl = pl.ds(i * 16, 16)
      mask_t[sl] = zf
      conf_t[sl] = zf + 1.0
      tx_t[sl] = zf
      ty_t[sl] = zf
      rw_t[sl] = zf + 1.0
      rh_t[sl] = zf + 1.0
      lab_t[sl] = zf + 127.0
      return c
    lax.fori_loop(0, _CELLS // 16, init_body, 0)

    def t_body(t, c):
      idx = t * 5 + jnp.where(in5, lanes, 0)
      row = plsc.load_gather(tgt_v, [idx], mask=in5)
      row = jnp.where(in5, row, 0.0)

      def ext(k):
        return jnp.sum(jnp.where(lanes == k, row, 0.0))

      def ifloor(v):
        r = v.astype(jnp.int32)
        return r - (r.astype(jnp.float32) > v).astype(jnp.int32)

      lab_f = ifloor(ext(0)).astype(jnp.float32)
      valid = jnp.sum(row) != 0.0
      gx = ext(1) * float(_GW)
      gy = ext(2) * float(_GH)
      gwv = ext(3) * float(_GW)
      ghv = ext(4) * float(_GH)
      gi = ifloor(gx)
      gj = ifloor(gy)
      inter = jnp.minimum(gwv, aw) * jnp.minimum(ghv, ah)
      union = gwv * ghv + aw * ah - inter + 1e-16
      iou = jnp.where(in5, inter / union, -1.0)
      mx = jnp.max(iou)
      best = jnp.min(jnp.where(iou == mx, lanes, 99))
      is_best = lanes == best
      high = iou > _THRESH
      cell = gj * _GW + gi

      conf_idx = jnp.where(in5, lanes, 0) * _HW + cell
      conf_val = jnp.where(is_best, 1.0, 0.0)
      conf_m = in5 & valid & (high | is_best)
      plsc.store_scatter(conf_t, [conf_idx], conf_val, mask=conf_m)

      aw_best = jnp.sum(jnp.where(is_best, aw, 0.0))
      ah_best = jnp.sum(jnp.where(is_best, ah, 0.0))
      bidx = (best * _HW + cell) + jnp.zeros((16,), jnp.int32)
      m0 = (lanes == 0) & valid
      plsc.store_scatter(mask_t, [bidx], zf + 1.0, mask=m0)
      plsc.store_scatter(tx_t, [bidx], zf + (gx - gi.astype(jnp.float32)),
                         mask=m0)
      plsc.store_scatter(ty_t, [bidx], zf + (gy - gj.astype(jnp.float32)),
                         mask=m0)
      plsc.store_scatter(rw_t, [bidx], (zf + gwv) / (zf + aw_best), mask=m0)
      plsc.store_scatter(rh_t, [bidx], (zf + ghv) / (zf + ah_best), mask=m0)
      old = plsc.load_gather(lab_t, [bidx], mask=m0)
      plsc.store_scatter(lab_t, [bidx], jnp.minimum(old, zf + lab_f), mask=m0)
      return c
    lax.fori_loop(0, _T, t_body, 0)

    pltpu.sync_copy(mask_t, mask_o.at[b])
    pltpu.sync_copy(conf_t, conf_o.at[b])
    pltpu.sync_copy(tx_t, tx_o.at[b])
    pltpu.sync_copy(ty_t, ty_o.at[b])
    pltpu.sync_copy(rw_t, rw_o.at[b])
    pltpu.sync_copy(rh_t, rh_o.at[b])
    pltpu.sync_copy(lab_t, lab_o.at[b])


@jax.jit
def _sc_build(tgt_pad):
  tab = jax.ShapeDtypeStruct((_B, _CELLS), jnp.float32)
  f = pl.kernel(
      _sc_body,
      out_type=(tab,) * 7,
      mesh=plsc.VectorSubcoreMesh(core_axis_name="c", subcore_axis_name="s",
                                  num_cores=_SC_CORES,
                                  num_subcores=_SC_SUBCORES),
      scratch_types=[pltpu.VMEM((_TPAD,), jnp.float32)] +
                    [pltpu.VMEM((_CELLS,), jnp.float32)] * 7,
      compiler_params=pltpu.CompilerParams(needs_layout_passes=False),
  )
  return f(tgt_pad)


def _sigmoid(z):
  return 1.0 / (1.0 + jnp.exp(-z))


def _tc_body(x_ref, mask_ref, conf_ref, tx_ref, ty_ref, rw_ref, rh_ref,
             lab_ref, out_ref, acc_ref):
  b = pl.program_id(0)

  @pl.when(b == 0)
  def _init():
    for i in range(6):
      acc_ref[i] = 0.0

  xb = x_ref[0]
  sq = 0.0
  bce_obj = 0.0
  bce_noobj = 0.0
  cls_s = 0.0
  n_obj = 0.0
  n_noobj = 0.0
  for a in range(_NA):
    sl = pl.ds(a * _HW, _HW)
    mask_v = mask_ref[0, 0, sl]
    conf_v = conf_ref[0, 0, sl]
    obj = mask_v > 0.5
    noobj_v = jnp.where(conf_v != mask_v, 1.0, 0.0)

    px = _sigmoid(xb[a * 14 + 0])
    py = _sigmoid(xb[a * 14 + 1])
    pw = xb[a * 14 + 2]
    ph = xb[a * 14 + 3]
    pconf = _sigmoid(xb[a * 14 + 6])

    twv = jnp.log(rw_ref[0, 0, sl] + 1e-16)
    thv = jnp.log(rh_ref[0, 0, sl] + 1e-16)
    d = ((px - tx_ref[0, 0, sl]) ** 2 + (py - ty_ref[0, 0, sl]) ** 2 +
         (pw - twv) ** 2 + (ph - thv) ** 2)
    sq += jnp.sum(jnp.where(obj, d, 0.0))

    p = jnp.clip(pconf, 1e-12, 1.0 - 1e-12)
    tcf = mask_v
    bce = -(tcf * jnp.log(p) + (1.0 - tcf) * jnp.log(1.0 - p))
    bce_obj += jnp.sum(jnp.where(obj, bce, 0.0))
    bce_noobj += jnp.sum(noobj_v * bce)

    s = [_sigmoid(xb[a * 14 + 7 + cc]) for cc in range(_NCLS)]
    m = s[0]
    for cc in range(1, _NCLS):
      m = jnp.maximum(m, s[cc])
    sumexp = jnp.exp(s[0] - m)
    for cc in range(1, _NCLS):
      sumexp += jnp.exp(s[cc] - m)
    lse = jnp.log(sumexp) + m
    lab_v = lab_ref[0, 0, sl]
    picked = -lse
    for cc in range(_NCLS):
      picked += jnp.where(lab_v == float(cc), s[cc], 0.0)
    cls_s += jnp.sum(jnp.where(obj, -picked, 0.0))

    n_obj += jnp.sum(mask_v)
    n_noobj += jnp.sum(noobj_v)

  acc_ref[0] += sq
  acc_ref[1] += bce_obj
  acc_ref[2] += bce_noobj
  acc_ref[3] += cls_s
  acc_ref[4] += n_obj
  acc_ref[5] += n_noobj

  @pl.when(b == pl.num_programs(0) - 1)
  def _fin():
    no = acc_ref[4]
    nn = acc_ref[5]
    out_ref[0, 0] = (acc_ref[0] / no + acc_ref[1] / no + acc_ref[2] / nn +
                     (1.0 / float(_B)) * acc_ref[3] / no)


@jax.jit
def _tc_loss(x3, mask, conf, tx, ty, rw, rh, lab):
  tab_spec = pl.BlockSpec((1, 1, _CELLS), lambda b: (b, 0, 0))
  return pl.pallas_call(
      _tc_body,
      grid=(_B,),
      in_specs=[pl.BlockSpec((1, 14 * _NA, _HW), lambda b: (b, 0, 0))] +
               [tab_spec] * 7,
      out_specs=pl.BlockSpec(memory_space=pltpu.SMEM),
      out_shape=jax.ShapeDtypeStruct((1, 1), jnp.float32),
      scratch_shapes=[pltpu.SMEM((6,), jnp.float32)],
  )(x3, mask, conf, tx, ty, rw, rh, lab)


def kernel(x, targets):
  tgt_pad = jnp.pad(jnp.reshape(targets, (_B, _T * 5)),
                    ((0, 0), (0, _TPAD - _T * 5)))
  tabs = _sc_build(tgt_pad)
  tabs3 = [jnp.reshape(t, (_B, 1, _CELLS)) for t in tabs]
  x3 = jnp.reshape(x, (_B, 14 * _NA, _HW))
  out = _tc_loss(x3, *tabs3)
  return jnp.reshape(out, ())


# speedup vs baseline: 109.1488x; 2.1892x over previous
import functools

import jax
import jax.numpy as jnp
import numpy as np
from jax import lax
from jax.experimental import pallas as pl
from jax.experimental.pallas import tpu as pltpu
from jax.experimental.pallas import tpu_sc as plsc

_ANCHORS = np.array(
    [[1.3221, 1.73145], [3.19275, 4.00944], [5.05587, 8.09892],
     [9.47112, 4.84053], [11.2364, 10.0071]], dtype=np.float32)
_THRESH = 0.6
_GH, _GW = 16, 32
_NA, _NCLS = 5, 7
_B, _T = 64, 50
_CELLS = _NA * _GH * _GW
_HW = _GH * _GW
_SC_CORES, _SC_SUBCORES = 2, 16
_NW = _SC_CORES * _SC_SUBCORES
_TPAD = 256


def _sc_body(tgt_hbm, mask_o, conf_o, tx_o, ty_o, rw_o, rh_o, lab_o,
             tgt_v, mask_t, conf_t, tx_t, ty_t, rw_t, rh_t, lab_t):
  wid = lax.axis_index("s") * _SC_CORES + lax.axis_index("c")
  lanes = lax.iota(jnp.int32, 16)
  in5 = lanes < _NA
  zf = jnp.zeros((16,), jnp.float32)

  def _lanes_const(vals):
    v = zf
    for i, c in enumerate(vals):
      v = jnp.where(lanes == i, float(c), v)
    return v

  aw = _lanes_const(_ANCHORS[:, 0])
  ah = _lanes_const(_ANCHORS[:, 1])

  for b_off in (0, _NW):
    b = wid + b_off
    pltpu.sync_copy(tgt_hbm.at[b], tgt_v)

    def init_body(i, c):
      sl = pl.ds(i * 16, 16)
      mask_t[sl] = zf
      conf_t[sl] = zf + 1.0
      tx_t[sl] = zf
      ty_t[sl] = zf
      rw_t[sl] = zf + 1.0
      rh_t[sl] = zf + 1.0
      lab_t[sl] = zf + 127.0
      return c
    lax.fori_loop(0, _CELLS // 16, init_body, 0)

    def t_body(t, c):
      idx = t * 5 + jnp.where(in5, lanes, 0)
      row = plsc.load_gather(tgt_v, [idx], mask=in5)
      row = jnp.where(in5, row, 0.0)

      def ext(k):
        return jnp.sum(jnp.where(lanes == k, row, 0.0))

      def ifloor(v):
        r = v.astype(jnp.int32)
        return r - (r.astype(jnp.float32) > v).astype(jnp.int32)

      lab_f = ifloor(ext(0)).astype(jnp.float32)
      valid = jnp.sum(row) != 0.0
      gx = ext(1) * float(_GW)
      gy = ext(2) * float(_GH)
      gwv = ext(3) * float(_GW)
      ghv = ext(4) * float(_GH)
      gi = ifloor(gx)
      gj = ifloor(gy)
      inter = jnp.minimum(gwv, aw) * jnp.minimum(ghv, ah)
      union = gwv * ghv + aw * ah - inter + 1e-16
      iou = jnp.where(in5, inter / union, -1.0)
      mx = jnp.max(iou)
      best = jnp.min(jnp.where(iou == mx, lanes, 99))
      is_best = lanes == best
      high = iou > _THRESH
      cell = gj * _GW + gi

      conf_idx = jnp.where(in5, lanes, 0) * _HW + cell
      conf_val = jnp.where(is_best, 1.0, 0.0)
      conf_m = in5 & valid & (high | is_best)
      plsc.store_scatter(conf_t, [conf_idx], conf_val, mask=conf_m)

      aw_best = jnp.sum(jnp.where(is_best, aw, 0.0))
      ah_best = jnp.sum(jnp.where(is_best, ah, 0.0))
      bidx = (best * _HW + cell) + jnp.zeros((16,), jnp.int32)
      m0 = (lanes == 0) & valid
      plsc.store_scatter(mask_t, [bidx], zf + 1.0, mask=m0)
      plsc.store_scatter(tx_t, [bidx], zf + (gx - gi.astype(jnp.float32)),
                         mask=m0)
      plsc.store_scatter(ty_t, [bidx], zf + (gy - gj.astype(jnp.float32)),
                         mask=m0)
      plsc.store_scatter(rw_t, [bidx], (zf + gwv) / (zf + aw_best), mask=m0)
      plsc.store_scatter(rh_t, [bidx], (zf + ghv) / (zf + ah_best), mask=m0)
      old = plsc.load_gather(lab_t, [bidx], mask=m0)
      plsc.store_scatter(lab_t, [bidx], jnp.minimum(old, zf + lab_f), mask=m0)
      return c
    lax.fori_loop(0, _T, t_body, 0)

    pltpu.sync_copy(mask_t, mask_o.at[b])
    pltpu.sync_copy(conf_t, conf_o.at[b])
    pltpu.sync_copy(tx_t, tx_o.at[b])
    pltpu.sync_copy(ty_t, ty_o.at[b])
    pltpu.sync_copy(rw_t, rw_o.at[b])
    pltpu.sync_copy(rh_t, rh_o.at[b])
    pltpu.sync_copy(lab_t, lab_o.at[b])


@jax.jit
def _sc_build(tgt_pad):
  tab = jax.ShapeDtypeStruct((_B, _CELLS), jnp.float32)
  f = pl.kernel(
      _sc_body,
      out_type=(tab,) * 7,
      mesh=plsc.VectorSubcoreMesh(core_axis_name="c", subcore_axis_name="s",
                                  num_cores=_SC_CORES,
                                  num_subcores=_SC_SUBCORES),
      scratch_types=[pltpu.VMEM((_TPAD,), jnp.float32)] +
                    [pltpu.VMEM((_CELLS,), jnp.float32)] * 7,
      compiler_params=pltpu.CompilerParams(needs_layout_passes=False),
  )
  return f(tgt_pad)


def _sigmoid(z):
  return 1.0 / (1.0 + jnp.exp(-z))


_BCHUNK = 16


def _tc_body(x_ref, mask_ref, conf_ref, tx_ref, ty_ref, rw_ref, rh_ref,
             lab_ref, out_ref, acc_ref):
  b = pl.program_id(0)

  @pl.when(b == 0)
  def _init():
    for i in range(6):
      acc_ref[i] = 0.0

  sq = 0.0
  bce_obj = 0.0
  bce_noobj = 0.0
  cls_s = 0.0
  n_obj = 0.0
  n_noobj = 0.0
  for a in range(_NA):
    sl = pl.ds(a * _HW, _HW)
    mask_v = mask_ref[:, sl]
    conf_v = conf_ref[:, sl]
    obj = mask_v > 0.5
    noobj_v = jnp.where(conf_v != mask_v, 1.0, 0.0)

    px = _sigmoid(x_ref[:, a * 14 + 0, :])
    py = _sigmoid(x_ref[:, a * 14 + 1, :])
    pw = x_ref[:, a * 14 + 2, :]
    ph = x_ref[:, a * 14 + 3, :]
    pconf = _sigmoid(x_ref[:, a * 14 + 6, :])

    twv = jnp.log(rw_ref[:, sl] + 1e-16)
    thv = jnp.log(rh_ref[:, sl] + 1e-16)
    d = ((px - tx_ref[:, sl]) ** 2 + (py - ty_ref[:, sl]) ** 2 +
         (pw - twv) ** 2 + (ph - thv) ** 2)
    sq += jnp.sum(jnp.where(obj, d, 0.0))

    p = jnp.clip(pconf, 1e-12, 1.0 - 1e-12)
    tcf = mask_v
    bce = -(tcf * jnp.log(p) + (1.0 - tcf) * jnp.log(1.0 - p))
    bce_obj += jnp.sum(jnp.where(obj, bce, 0.0))
    bce_noobj += jnp.sum(noobj_v * bce)

    s = [_sigmoid(x_ref[:, a * 14 + 7 + cc, :]) for cc in range(_NCLS)]
    m = s[0]
    for cc in range(1, _NCLS):
      m = jnp.maximum(m, s[cc])
    sumexp = jnp.exp(s[0] - m)
    for cc in range(1, _NCLS):
      sumexp += jnp.exp(s[cc] - m)
    lse = jnp.log(sumexp) + m
    lab_v = lab_ref[:, sl]
    picked = -lse
    for cc in range(_NCLS):
      picked += jnp.where(lab_v == float(cc), s[cc], 0.0)
    cls_s += jnp.sum(jnp.where(obj, -picked, 0.0))

    n_obj += jnp.sum(mask_v)
    n_noobj += jnp.sum(noobj_v)

  acc_ref[0] += sq
  acc_ref[1] += bce_obj
  acc_ref[2] += bce_noobj
  acc_ref[3] += cls_s
  acc_ref[4] += n_obj
  acc_ref[5] += n_noobj

  @pl.when(b == pl.num_programs(0) - 1)
  def _fin():
    no = acc_ref[4]
    nn = acc_ref[5]
    out_ref[0, 0] = (acc_ref[0] / no + acc_ref[1] / no + acc_ref[2] / nn +
                     (1.0 / float(_B)) * acc_ref[3] / no)


@jax.jit
def _tc_loss(x3, mask, conf, tx, ty, rw, rh, lab):
  tab_spec = pl.BlockSpec((_BCHUNK, _CELLS), lambda b: (b, 0))
  return pl.pallas_call(
      _tc_body,
      grid=(_B // _BCHUNK,),
      in_specs=[pl.BlockSpec((_BCHUNK, 14 * _NA, _HW), lambda b: (b, 0, 0))] +
               [tab_spec] * 7,
      out_specs=pl.BlockSpec(memory_space=pltpu.SMEM),
      out_shape=jax.ShapeDtypeStruct((1, 1), jnp.float32),
      scratch_shapes=[pltpu.SMEM((6,), jnp.float32)],
  )(x3, mask, conf, tx, ty, rw, rh, lab)


def kernel(x, targets):
  tgt_pad = jnp.pad(jnp.reshape(targets, (_B, _T * 5)),
                    ((0, 0), (0, _TPAD - _T * 5)))
  tabs = _sc_build(tgt_pad)
  x3 = jnp.reshape(x, (_B, 14 * _NA, _HW))
  out = _tc_loss(x3, *tabs)
  return jnp.reshape(out, ())


# speedup vs baseline: 109.2182x; 1.0006x over previous
import functools

import jax
import jax.numpy as jnp
import numpy as np
from jax import lax
from jax.experimental import pallas as pl
from jax.experimental.pallas import tpu as pltpu
from jax.experimental.pallas import tpu_sc as plsc

_ANCHORS = np.array(
    [[1.3221, 1.73145], [3.19275, 4.00944], [5.05587, 8.09892],
     [9.47112, 4.84053], [11.2364, 10.0071]], dtype=np.float32)
_THRESH = 0.6
_GH, _GW = 16, 32
_NA, _NCLS = 5, 7
_B, _T = 64, 50
_CELLS = _NA * _GH * _GW
_HW = _GH * _GW
_SC_CORES, _SC_SUBCORES = 2, 16
_NW = _SC_CORES * _SC_SUBCORES
_TPAD = 256


def _sc_body(tgt_hbm, mask_o, conf_o, tx_o, ty_o, rw_o, rh_o, lab_o,
             tgt_v, mask_t, conf_t, tx_t, ty_t, rw_t, rh_t, lab_t):
  wid = lax.axis_index("s") * _SC_CORES + lax.axis_index("c")
  lanes = lax.iota(jnp.int32, 16)
  in5 = lanes < _NA
  zf = jnp.zeros((16,), jnp.float32)

  def _lanes_const(vals):
    v = zf
    for i, c in enumerate(vals):
      v = jnp.where(lanes == i, float(c), v)
    return v

  aw = _lanes_const(_ANCHORS[:, 0])
  ah = _lanes_const(_ANCHORS[:, 1])

  for b_off in (0, _NW):
    b = wid + b_off
    pltpu.sync_copy(tgt_hbm.at[b], tgt_v)

    def init_body(i, c):
      sl = pl.ds(i * 16, 16)
      mask_t[sl] = zf
      conf_t[sl] = zf + 1.0
      tx_t[sl] = zf
      ty_t[sl] = zf
      rw_t[sl] = zf + 1.0
      rh_t[sl] = zf + 1.0
      lab_t[sl] = zf + 127.0
      return c
    lax.fori_loop(0, _CELLS // 16, init_body, 0)

    def t_body(t, c):
      idx = t * 5 + jnp.where(in5, lanes, 0)
      row = plsc.load_gather(tgt_v, [idx], mask=in5)
      row = jnp.where(in5, row, 0.0)

      def ext(k):
        return jnp.sum(jnp.where(lanes == k, row, 0.0))

      def ifloor(v):
        r = v.astype(jnp.int32)
        return r - (r.astype(jnp.float32) > v).astype(jnp.int32)

      lab_f = ifloor(ext(0)).astype(jnp.float32)
      valid = jnp.sum(row) != 0.0
      gx = ext(1) * float(_GW)
      gy = ext(2) * float(_GH)
      gwv = ext(3) * float(_GW)
      ghv = ext(4) * float(_GH)
      gi = ifloor(gx)
      gj = ifloor(gy)
      inter = jnp.minimum(gwv, aw) * jnp.minimum(ghv, ah)
      union = gwv * ghv + aw * ah - inter + 1e-16
      iou = jnp.where(in5, inter / union, -1.0)
      mx = jnp.max(iou)
      best = jnp.min(jnp.where(iou == mx, lanes, 99))
      is_best = lanes == best
      high = iou > _THRESH
      cell = gj * _GW + gi

      conf_idx = jnp.where(in5, lanes, 0) * _HW + cell
      conf_val = jnp.where(is_best, 1.0, 0.0)
      conf_m = in5 & valid & (high | is_best)
      plsc.store_scatter(conf_t, [conf_idx], conf_val, mask=conf_m)

      aw_best = jnp.sum(jnp.where(is_best, aw, 0.0))
      ah_best = jnp.sum(jnp.where(is_best, ah, 0.0))
      bidx = (best * _HW + cell) + jnp.zeros((16,), jnp.int32)
      m0 = (lanes == 0) & valid
      plsc.store_scatter(mask_t, [bidx], zf + 1.0, mask=m0)
      plsc.store_scatter(tx_t, [bidx], zf + (gx - gi.astype(jnp.float32)),
                         mask=m0)
      plsc.store_scatter(ty_t, [bidx], zf + (gy - gj.astype(jnp.float32)),
                         mask=m0)
      plsc.store_scatter(rw_t, [bidx], (zf + gwv) / (zf + aw_best), mask=m0)
      plsc.store_scatter(rh_t, [bidx], (zf + ghv) / (zf + ah_best), mask=m0)
      old = plsc.load_gather(lab_t, [bidx], mask=m0)
      plsc.store_scatter(lab_t, [bidx], jnp.minimum(old, zf + lab_f), mask=m0)
      return c
    lax.fori_loop(0, _T, t_body, 0)

    pltpu.sync_copy(mask_t, mask_o.at[b])
    pltpu.sync_copy(conf_t, conf_o.at[b])
    pltpu.sync_copy(tx_t, tx_o.at[b])
    pltpu.sync_copy(ty_t, ty_o.at[b])
    pltpu.sync_copy(rw_t, rw_o.at[b])
    pltpu.sync_copy(rh_t, rh_o.at[b])
    pltpu.sync_copy(lab_t, lab_o.at[b])


@jax.jit
def _sc_build(tgt_pad):
  tab = jax.ShapeDtypeStruct((_B, _CELLS), jnp.float32)
  f = pl.kernel(
      _sc_body,
      out_type=(tab,) * 7,
      mesh=plsc.VectorSubcoreMesh(core_axis_name="c", subcore_axis_name="s",
                                  num_cores=_SC_CORES,
                                  num_subcores=_SC_SUBCORES),
      scratch_types=[pltpu.VMEM((_TPAD,), jnp.float32)] +
                    [pltpu.VMEM((_CELLS,), jnp.float32)] * 7,
      compiler_params=pltpu.CompilerParams(needs_layout_passes=False),
  )
  return f(tgt_pad)


def _sigmoid(z):
  return 1.0 / (1.0 + jnp.exp(-z))


_BCHUNK = 16


def _tc_body(x_ref, mask_ref, conf_ref, tx_ref, ty_ref, rw_ref, rh_ref,
             lab_ref, out_ref, acc_ref):
  b = pl.program_id(0)

  @pl.when(b == 0)
  def _init():
    for i in range(6):
      acc_ref[i] = 0.0

  sq = 0.0
  bce_obj = 0.0
  bce_noobj = 0.0
  cls_s = 0.0
  n_obj = 0.0
  n_noobj = 0.0
  for a in range(_NA):
    sl = pl.ds(a * _HW, _HW)
    mask_v = mask_ref[:, sl]
    conf_v = conf_ref[:, sl]
    obj = mask_v > 0.5
    noobj_v = jnp.where(conf_v != mask_v, 1.0, 0.0)

    px = _sigmoid(x_ref[:, a * 14 + 0, :])
    py = _sigmoid(x_ref[:, a * 14 + 1, :])
    pw = x_ref[:, a * 14 + 2, :]
    ph = x_ref[:, a * 14 + 3, :]
    pconf = _sigmoid(x_ref[:, a * 14 + 6, :])

    twv = jnp.log(rw_ref[:, sl] + 1e-16)
    thv = jnp.log(rh_ref[:, sl] + 1e-16)
    d = ((px - tx_ref[:, sl]) ** 2 + (py - ty_ref[:, sl]) ** 2 +
         (pw - twv) ** 2 + (ph - thv) ** 2)
    sq += jnp.sum(jnp.where(obj, d, 0.0))

    p = jnp.clip(pconf, 1e-12, 1.0 - 1e-12)
    tcf = mask_v
    bce = -(tcf * jnp.log(p) + (1.0 - tcf) * jnp.log(1.0 - p))
    bce_obj += jnp.sum(jnp.where(obj, bce, 0.0))
    bce_noobj += jnp.sum(noobj_v * bce)

    s = [_sigmoid(x_ref[:, a * 14 + 7 + cc, :]) for cc in range(_NCLS)]
    m = s[0]
    for cc in range(1, _NCLS):
      m = jnp.maximum(m, s[cc])
    sumexp = jnp.exp(s[0] - m)
    for cc in range(1, _NCLS):
      sumexp += jnp.exp(s[cc] - m)
    lse = jnp.log(sumexp) + m
    lab_v = lab_ref[:, sl]
    picked = -lse
    for cc in range(_NCLS):
      picked += jnp.where(lab_v == float(cc), s[cc], 0.0)
    cls_s += jnp.sum(jnp.where(obj, -picked, 0.0))

    n_obj += jnp.sum(mask_v)
    n_noobj += jnp.sum(noobj_v)

  acc_ref[0] += sq
  acc_ref[1] += bce_obj
  acc_ref[2] += bce_noobj
  acc_ref[3] += cls_s
  acc_ref[4] += n_obj
  acc_ref[5] += n_noobj

  @pl.when(b == pl.num_programs(0) - 1)
  def _fin():
    no = acc_ref[4]
    nn = acc_ref[5]
    out_ref[0, 0] = (acc_ref[0] / no + acc_ref[1] / no + acc_ref[2] / nn +
                     (1.0 / float(_B)) * acc_ref[3] / no)


@jax.jit
def _tc_loss(x3, mask, conf, tx, ty, rw, rh, lab):
  tab_spec = pl.BlockSpec((_BCHUNK, _CELLS), lambda b: (b, 0))
  return pl.pallas_call(
      _tc_body,
      grid=(_B // _BCHUNK,),
      in_specs=[pl.BlockSpec((_BCHUNK, 14 * _NA, _HW), lambda b: (b, 0, 0))] +
               [tab_spec] * 7,
      out_specs=pl.BlockSpec(memory_space=pltpu.SMEM),
      out_shape=jax.ShapeDtypeStruct((1, 1), jnp.float32),
      scratch_shapes=[pltpu.SMEM((6,), jnp.float32)],
  )(x3, mask, conf, tx, ty, rw, rh, lab)


@jax.jit
def kernel(x, targets):
  tgt_pad = jnp.pad(jnp.reshape(targets, (_B, _T * 5)),
                    ((0, 0), (0, _TPAD - _T * 5)))
  tabs = _sc_build(tgt_pad)
  x3 = jnp.reshape(x, (_B, 14 * _NA, _HW))
  out = _tc_loss(x3, *tabs)
  return jnp.reshape(out, ())
